# baseline (speedup 1.0000x reference)
import jax
import jax.numpy as jnp
from jax.experimental import pallas as pl
from jax.experimental.pallas import tpu as pltpu

_W = 18432


def _seg_max_body(s_ref, x_ref, o_ref):
    xb = x_ref[...]
    tio = jax.lax.broadcasted_iota(jnp.int32, xb.shape, 0)
    n = o_ref.shape[0]
    neg = jnp.float32(-jnp.inf)
    for i in range(n):
        m = (tio >= s_ref[i]) & (tio < s_ref[n + i])
        o_ref[i, :] = jnp.max(jnp.where(m, xb, neg), axis=0)


def kernel(x, record_len):
    T = x.shape[0]
    n = record_len.shape[0]
    S = 1
    for d in x.shape[1:]:
        S *= d
    xr = x.reshape(T, S)

    cs = jnp.cumsum(record_len.astype(jnp.int32))
    starts = jnp.concatenate([jnp.zeros((1,), jnp.int32), cs[:-1]])
    ends = jnp.concatenate([cs[:-1], jnp.full((1,), T, jnp.int32)])
    bounds = jnp.concatenate([starts, ends])

    grid = S // _W
    out = pl.pallas_call(
        _seg_max_body,
        grid_spec=pltpu.PrefetchScalarGridSpec(
            num_scalar_prefetch=1,
            grid=(grid,),
            in_specs=[pl.BlockSpec((T, _W), lambda j, s: (0, j))],
            out_specs=pl.BlockSpec((n, _W), lambda j, s: (0, j)),
        ),
        out_shape=jax.ShapeDtypeStruct((n, S), jnp.float32),
    )(bounds, xr)
    return out.reshape((n,) + x.shape[1:])


# speedup vs baseline: 14.1751x; 14.1751x over previous
import jax
import jax.numpy as jnp
from jax.experimental import pallas as pl
from jax.experimental.pallas import tpu as pltpu

_BC = 4


def _seg_max_body(s_ref, x_ref, o_ref):
    xb = x_ref[...]
    tio = jax.lax.broadcasted_iota(jnp.int32, xb.shape, 0)
    n = o_ref.shape[0]
    neg = jnp.float32(-jnp.inf)
    for i in range(n):
        m = (tio >= s_ref[i]) & (tio < s_ref[n + i])
        o_ref[i] = jnp.max(jnp.where(m, xb, neg), axis=0)


def kernel(x, record_len):
    T, C, H, W = x.shape
    n = record_len.shape[0]

    cs = jnp.cumsum(record_len.astype(jnp.int32))
    starts = jnp.concatenate([jnp.zeros((1,), jnp.int32), cs[:-1]])
    ends = jnp.concatenate([cs[:-1], jnp.full((1,), T, jnp.int32)])
    bounds = jnp.concatenate([starts, ends])

    grid = C // _BC
    return pl.pallas_call(
        _seg_max_body,
        grid_spec=pltpu.PrefetchScalarGridSpec(
            num_scalar_prefetch=1,
            grid=(grid,),
            in_specs=[pl.BlockSpec((T, _BC, H, W), lambda j, s: (0, j, 0, 0))],
            out_specs=pl.BlockSpec((n, _BC, H, W), lambda j, s: (0, j, 0, 0)),
        ),
        out_shape=jax.ShapeDtypeStruct((n, C, H, W), jnp.float32),
    )(bounds, x)


# speedup vs baseline: 14.7419x; 1.0400x over previous
import jax
import jax.numpy as jnp
from jax.experimental import pallas as pl
from jax.experimental.pallas import tpu as pltpu

_BC = 4


def _seg_max_body(s_ref, x_ref, o_ref):
    n = o_ref.shape[0]
    neg = jnp.float32(-jnp.inf)
    for i in range(n):
        s = s_ref[i]
        e = s_ref[n + i]
        row0 = x_ref[jnp.minimum(s, x_ref.shape[0] - 1)]
        o_ref[i] = jnp.where(e > s, row0, jnp.full_like(row0, neg))

        def acc(t, c):
            o_ref[i] = jnp.maximum(o_ref[i], x_ref[t])
            return c

        jax.lax.fori_loop(s + 1, e, acc, 0)


def kernel(x, record_len):
    T, C, H, W = x.shape
    n = record_len.shape[0]

    cs = jnp.cumsum(record_len.astype(jnp.int32))
    starts = jnp.concatenate([jnp.zeros((1,), jnp.int32), cs[:-1]])
    ends = jnp.concatenate([cs[:-1], jnp.full((1,), T, jnp.int32)])
    starts = jnp.clip(starts, 0, T)
    ends = jnp.clip(ends, 0, T)
    bounds = jnp.concatenate([starts, ends])

    grid = C // _BC
    return pl.pallas_call(
        _seg_max_body,
        grid_spec=pltpu.PrefetchScalarGridSpec(
            num_scalar_prefetch=1,
            grid=(grid,),
            in_specs=[pl.BlockSpec((T, _BC, H, W), lambda j, s: (0, j, 0, 0))],
            out_specs=pl.BlockSpec((n, _BC, H, W), lambda j, s: (0, j, 0, 0)),
        ),
        out_shape=jax.ShapeDtypeStruct((n, C, H, W), jnp.float32),
    )(bounds, x)
